# baseline (speedup 1.0000x reference)
import jax
import jax.numpy as jnp
from jax import lax
from jax.experimental import pallas as pl
from jax.experimental.pallas import tpu as pltpu
from jax.experimental.pallas import tpu_sc as plsc

VOCAB = 100000
SEQ_LEN = 200
EMBED = 128
BATCH = 1024
EPS = 1e-5

NC = 2
NS = 16
NW = NC * NS
N_ROWS = BATCH * SEQ_LEN
NSLAB = 2
BATCH_S = BATCH // NSLAB
ROWS_S = BATCH_S * SEQ_LEN
ROWS_PER_TILE = ROWS_S // NW
CHUNK = 32
K = ROWS_PER_TILE // CHUNK
NBUF = 4

TC_B = 64
TC_GRID_S = BATCH_S // TC_B


def _sc_gather_body(ids_hbm, table_hbm, out_hbm, idx_v, *scratch):
    bufs = scratch[:NBUF]
    gsems = scratch[NBUF:2 * NBUF]
    osems = scratch[2 * NBUF:3 * NBUF]

    wid = lax.axis_index("s") * NC + lax.axis_index("c")
    base_row = wid * ROWS_PER_TILE

    pltpu.sync_copy(ids_hbm.at[wid], idx_v)

    def fire_gather(k, j):
        pltpu.async_copy(table_hbm.at[idx_v.at[k]], bufs[j], gsems[j])

    def wait_gather(k, j):
        pltpu.make_async_copy(table_hbm.at[idx_v.at[k]], bufs[j], gsems[j]).wait()

    def fire_scatter(k, j):
        pltpu.async_copy(
            bufs[j], out_hbm.at[pl.ds(base_row + k * CHUNK, CHUNK)], osems[j])

    def wait_scatter(k, j):
        pltpu.make_async_copy(
            bufs[j], out_hbm.at[pl.ds(base_row + k * CHUNK, CHUNK)],
            osems[j]).wait()

    for j in range(NBUF):
        fire_gather(j, j)

    @pl.loop(0, K, step=NBUF)
    def _chunk(k):
        for j in range(NBUF):
            wait_gather(k + j, j)
            fire_scatter(k + j, j)
        for j in range(NBUF):
            wait_scatter(k + j, j)

            @pl.when(k + NBUF + j < K)
            def _():
                fire_gather(k + NBUF + j, j)


def _tc_ln_body(x_ref, pos_ref, g_ref, b_ref, o_ref):
    t = x_ref[...] + pos_ref[...][None, :, :]
    mean = jnp.mean(t, axis=-1, keepdims=True)
    c = t - mean
    var = jnp.mean(c * c, axis=-1, keepdims=True)
    rstd = lax.rsqrt(var + EPS)
    o_ref[...] = c * rstd * g_ref[...] + b_ref[...]


def _tc_ln_body_alias(x_ref, pos_ref, g_ref, b_ref, acc_ref, o_ref):
    del acc_ref
    _tc_ln_body(x_ref, pos_ref, g_ref, b_ref, o_ref)


@jax.jit
def _run(ids4d, token_table, pos_table, gamma, beta):
    mesh = plsc.VectorSubcoreMesh(core_axis_name="c", subcore_axis_name="s",
                                  num_cores=NC, num_subcores=NS)
    sc_gather = pl.kernel(
        _sc_gather_body,
        out_type=jax.ShapeDtypeStruct((ROWS_S, EMBED), jnp.float32),
        mesh=mesh,
        scratch_types=(
            [pltpu.VMEM((K, CHUNK), jnp.int32)]
            + [pltpu.VMEM((CHUNK, EMBED), jnp.float32) for _ in range(NBUF)]
            + [pltpu.SemaphoreType.DMA for _ in range(2 * NBUF)]
        ),
    )

    g0 = sc_gather(ids4d[0], token_table)
    g1 = sc_gather(ids4d[1], token_table)
    x0 = g0.reshape(BATCH_S, SEQ_LEN, EMBED)
    x1 = g1.reshape(BATCH_S, SEQ_LEN, EMBED)

    common_specs = [
        pl.BlockSpec((TC_B, SEQ_LEN, EMBED), lambda i: (i, 0, 0)),
        pl.BlockSpec((SEQ_LEN, EMBED), lambda i: (0, 0)),
        pl.BlockSpec((EMBED,), lambda i: (0,)),
        pl.BlockSpec((EMBED,), lambda i: (0,)),
    ]

    acc = pl.pallas_call(
        _tc_ln_body,
        out_shape=jax.ShapeDtypeStruct((BATCH, SEQ_LEN, EMBED), jnp.float32),
        grid=(TC_GRID_S,),
        in_specs=common_specs,
        out_specs=pl.BlockSpec((TC_B, SEQ_LEN, EMBED), lambda i: (i, 0, 0)),
    )(x0, pos_table, gamma, beta)

    out = pl.pallas_call(
        _tc_ln_body_alias,
        out_shape=jax.ShapeDtypeStruct((BATCH, SEQ_LEN, EMBED), jnp.float32),
        grid=(TC_GRID_S,),
        in_specs=common_specs + [
            pl.BlockSpec(memory_space=pltpu.MemorySpace.HBM)],
        out_specs=pl.BlockSpec((TC_B, SEQ_LEN, EMBED),
                               lambda i: (i + TC_GRID_S, 0, 0)),
        input_output_aliases={4: 0},
    )(x1, pos_table, gamma, beta, acc)
    return out


def kernel(input_ids, token_table, pos_table, gamma, beta):
    ids4d = jnp.reshape(input_ids.astype(jnp.int32), (NSLAB, NW, K, CHUNK))
    return _run(ids4d, token_table, pos_table, gamma, beta)


# speedup vs baseline: 1.0392x; 1.0392x over previous
import jax
import jax.numpy as jnp
from jax import lax
from jax.experimental import pallas as pl
from jax.experimental.pallas import tpu as pltpu
from jax.experimental.pallas import tpu_sc as plsc

VOCAB = 100000
SEQ_LEN = 200
EMBED = 128
BATCH = 1024
EPS = 1e-5

NC = 2
NS = 16
NW = NC * NS
N_ROWS = BATCH * SEQ_LEN
ROWS_PER_TILE = N_ROWS // NW
CHUNK = 64
K = ROWS_PER_TILE // CHUNK
NBUF = 4

TC_B = 64


def _sc_gather_body(ids_hbm, table_hbm, out_hbm, idx_v, *scratch):
    bufs = scratch[:NBUF]
    gsems = scratch[NBUF:2 * NBUF]
    osems = scratch[2 * NBUF:3 * NBUF]

    wid = lax.axis_index("s") * NC + lax.axis_index("c")
    base_row = wid * ROWS_PER_TILE

    pltpu.sync_copy(ids_hbm.at[wid], idx_v)

    def fire_gather(k, j):
        pltpu.async_copy(table_hbm.at[idx_v.at[k]], bufs[j], gsems[j])

    def wait_gather(k, j):
        pltpu.make_async_copy(table_hbm.at[idx_v.at[k]], bufs[j], gsems[j]).wait()

    def fire_scatter(k, j):
        pltpu.async_copy(
            bufs[j], out_hbm.at[pl.ds(base_row + k * CHUNK, CHUNK)], osems[j])

    def wait_scatter(k, j):
        pltpu.make_async_copy(
            bufs[j], out_hbm.at[pl.ds(base_row + k * CHUNK, CHUNK)],
            osems[j]).wait()

    for j in range(NBUF):
        fire_gather(j, j)

    @pl.loop(0, K, step=NBUF)
    def _chunk(k):
        for j in range(NBUF):
            wait_gather(k + j, j)
            fire_scatter(k + j, j)
        for j in range(NBUF):
            wait_scatter(k + j, j)

            @pl.when(k + NBUF + j < K)
            def _():
                fire_gather(k + NBUF + j, j)


def _tc_ln_body(x_ref, pos_ref, g_ref, b_ref, o_ref):
    t = x_ref[...].astype(jnp.float32) + pos_ref[...][None, :, :]
    mean = jnp.mean(t, axis=-1, keepdims=True)
    c = t - mean
    var = jnp.mean(c * c, axis=-1, keepdims=True)
    rstd = lax.rsqrt(var + EPS)
    o_ref[...] = c * rstd * g_ref[...] + b_ref[...]


@jax.jit
def _run(ids3d, table, pos_table, gamma, beta):
    mesh = plsc.VectorSubcoreMesh(core_axis_name="c", subcore_axis_name="s",
                                  num_cores=NC, num_subcores=NS)
    gathered = pl.kernel(
        _sc_gather_body,
        out_type=jax.ShapeDtypeStruct((N_ROWS, EMBED), jnp.float32),
        mesh=mesh,
        scratch_types=(
            [pltpu.VMEM((K, CHUNK), jnp.int32)]
            + [pltpu.VMEM((CHUNK, EMBED), jnp.float32) for _ in range(NBUF)]
            + [pltpu.SemaphoreType.DMA for _ in range(2 * NBUF)]
        ),
    )(ids3d, table)

    x = gathered.reshape(BATCH, SEQ_LEN, EMBED)
    out = pl.pallas_call(
        _tc_ln_body,
        out_shape=jax.ShapeDtypeStruct((BATCH, SEQ_LEN, EMBED), jnp.float32),
        grid=(BATCH // TC_B,),
        in_specs=[
            pl.BlockSpec((TC_B, SEQ_LEN, EMBED), lambda i: (i, 0, 0)),
            pl.BlockSpec((SEQ_LEN, EMBED), lambda i: (0, 0)),
            pl.BlockSpec((EMBED,), lambda i: (0,)),
            pl.BlockSpec((EMBED,), lambda i: (0,)),
        ],
        out_specs=pl.BlockSpec((TC_B, SEQ_LEN, EMBED), lambda i: (i, 0, 0)),
    )(x, pos_table, gamma, beta)
    return out


def kernel(input_ids, token_table, pos_table, gamma, beta):
    ids3d = jnp.reshape(input_ids.astype(jnp.int32), (NW, K, CHUNK))
    return _run(ids3d, token_table, pos_table, gamma, beta)


# speedup vs baseline: 1.0735x; 1.0330x over previous
import jax
import jax.numpy as jnp
from jax import lax
from jax.experimental import pallas as pl
from jax.experimental.pallas import tpu as pltpu
from jax.experimental.pallas import tpu_sc as plsc

VOCAB = 100000
SEQ_LEN = 200
EMBED = 128
BATCH = 1024
EPS = 1e-5

NC = 2
NS = 16
NW = NC * NS
HALF_B = BATCH // 2
ROWS_H = HALF_B * SEQ_LEN
ROWS_PER_TILE = ROWS_H // NW
CHUNK = 64
K = ROWS_PER_TILE // CHUNK
NBUF = 5

TC_B = 64
GRID_H = HALF_B // TC_B


def _sc_gather_body(ids_hbm, table_hbm, out_hbm, idx_v, *scratch):
    bufs = scratch[:NBUF]
    gsems = scratch[NBUF:2 * NBUF]
    osems = scratch[2 * NBUF:3 * NBUF]

    wid = lax.axis_index("s") * NC + lax.axis_index("c")
    base_row = wid * ROWS_PER_TILE

    pltpu.sync_copy(ids_hbm.at[wid], idx_v)

    def fire_gather(k, j):
        pltpu.async_copy(table_hbm.at[idx_v.at[k]], bufs[j], gsems[j])

    def wait_gather(k, j):
        pltpu.make_async_copy(table_hbm.at[idx_v.at[k]], bufs[j], gsems[j]).wait()

    def fire_scatter(k, j):
        pltpu.async_copy(
            bufs[j], out_hbm.at[pl.ds(base_row + k * CHUNK, CHUNK)], osems[j])

    def wait_scatter(k, j):
        pltpu.make_async_copy(
            bufs[j], out_hbm.at[pl.ds(base_row + k * CHUNK, CHUNK)],
            osems[j]).wait()

    for j in range(NBUF):
        fire_gather(j, j)

    @pl.loop(0, K, step=NBUF)
    def _chunk(k):
        for j in range(NBUF):
            wait_gather(k + j, j)
            fire_scatter(k + j, j)
        for j in range(NBUF):
            wait_scatter(k + j, j)

            @pl.when(k + NBUF + j < K)
            def _():
                fire_gather(k + NBUF + j, j)


def _tc_ln_body(x_ref, pos_ref, g_ref, b_ref, o_ref):
    t = x_ref[...] + pos_ref[...][None, :, :]
    mean = jnp.mean(t, axis=-1, keepdims=True)
    c = t - mean
    var = jnp.mean(c * c, axis=-1, keepdims=True)
    rstd = lax.rsqrt(var + EPS)
    o_ref[...] = c * rstd * g_ref[...] + b_ref[...]


def _tc_ln_body2(alias_ref, x_ref, pos_ref, g_ref, b_ref, o_ref):
    del alias_ref
    _tc_ln_body(x_ref, pos_ref, g_ref, b_ref, o_ref)


@jax.jit
def _run(ids_a, ids_b, table, pos_table, gamma, beta):
    mesh = plsc.VectorSubcoreMesh(core_axis_name="c", subcore_axis_name="s",
                                  num_cores=NC, num_subcores=NS)

    def gather(ids3d):
        return pl.kernel(
            _sc_gather_body,
            out_type=jax.ShapeDtypeStruct((ROWS_H, EMBED), jnp.float32),
            mesh=mesh,
            scratch_types=(
                [pltpu.VMEM((K, CHUNK), jnp.int32)]
                + [pltpu.VMEM((CHUNK, EMBED), jnp.float32) for _ in range(NBUF)]
                + [pltpu.SemaphoreType.DMA for _ in range(2 * NBUF)]
            ),
        )(ids3d, table)

    g0 = gather(ids_a).reshape(HALF_B, SEQ_LEN, EMBED)
    g1 = gather(ids_b).reshape(HALF_B, SEQ_LEN, EMBED)

    out0 = pl.pallas_call(
        _tc_ln_body,
        out_shape=jax.ShapeDtypeStruct((BATCH, SEQ_LEN, EMBED), jnp.float32),
        grid=(GRID_H,),
        in_specs=[
            pl.BlockSpec((TC_B, SEQ_LEN, EMBED), lambda i: (i, 0, 0)),
            pl.BlockSpec((SEQ_LEN, EMBED), lambda i: (0, 0)),
            pl.BlockSpec((EMBED,), lambda i: (0,)),
            pl.BlockSpec((EMBED,), lambda i: (0,)),
        ],
        out_specs=pl.BlockSpec((TC_B, SEQ_LEN, EMBED), lambda i: (i, 0, 0)),
    )(g0, pos_table, gamma, beta)

    out = pl.pallas_call(
        _tc_ln_body2,
        out_shape=jax.ShapeDtypeStruct((BATCH, SEQ_LEN, EMBED), jnp.float32),
        grid=(GRID_H,),
        in_specs=[
            pl.BlockSpec(memory_space=pl.ANY),
            pl.BlockSpec((TC_B, SEQ_LEN, EMBED), lambda i: (i, 0, 0)),
            pl.BlockSpec((SEQ_LEN, EMBED), lambda i: (0, 0)),
            pl.BlockSpec((EMBED,), lambda i: (0,)),
            pl.BlockSpec((EMBED,), lambda i: (0,)),
        ],
        out_specs=pl.BlockSpec((TC_B, SEQ_LEN, EMBED),
                               lambda i: (i + GRID_H, 0, 0)),
        input_output_aliases={0: 0},
    )(out0, g1, pos_table, gamma, beta)
    return out


def kernel(input_ids, token_table, pos_table, gamma, beta):
    ids = input_ids.astype(jnp.int32)
    ids_a = jnp.reshape(ids[:HALF_B], (NW, K, CHUNK))
    ids_b = jnp.reshape(ids[HALF_B:], (NW, K, CHUNK))
    return _run(ids_a, ids_b, token_table, pos_table, gamma, beta)
